# baseline (speedup 1.0000x reference)
import functools

import jax
import jax.numpy as jnp
from jax import lax
from jax.experimental import pallas as pl
from jax.experimental.pallas import tpu as pltpu
from jax.experimental.pallas import tpu_sc as plsc

N = 10000
D = 128
E = 320000

NC = 2
NS = 16
NW = NC * NS
C = 128
NCHUNK = 80
EPT = NCHUNK * C
EP = NW * EPT
NP = 10240
RPT = NP // NS

_mesh = plsc.VectorSubcoreMesh(core_axis_name="c", subcore_axis_name="s")


def _zero_fill(buf, rows, width):
    z = jnp.zeros((16,), jnp.float32)

    @pl.loop(0, rows)
    def _(i):
        for j in range(width // 16):
            buf[i, pl.ds(j * 16, 16)] = z


def _fill_ones(buf, rows, width):
    o = jnp.ones((16,), jnp.float32)

    @pl.loop(0, rows)
    def _(i):
        for j in range(width // 16):
            buf[i, pl.ds(j * 16, 16)] = o


@functools.partial(
    pl.kernel,
    out_type=jax.ShapeDtypeStruct((NC, NP, D), jnp.float32),
    mesh=_mesh,
    scratch_types=[
        pltpu.VMEM((NCHUNK, C), jnp.int32),
        pltpu.VMEM((C, D), jnp.float32),
        pltpu.VMEM_SHARED((NP, D), jnp.float32),
    ],
)
def _deg_sc(idx_hbm, deg_hbm, ix_v, ones_v, acc):
    c = lax.axis_index("c")
    s = lax.axis_index("s")
    wid = c * NS + s

    pltpu.sync_copy(idx_hbm.at[wid], ix_v)

    _zero_fill(ones_v, C, D)
    for t in range(RPT // C):
        pltpu.sync_copy(ones_v, acc.at[pl.ds(s * RPT + t * C, C)])
    _fill_ones(ones_v, C, D)
    plsc.subcore_barrier()

    @pl.loop(0, NCHUNK)
    def _(k):
        pltpu.sync_copy(ones_v, acc.at[ix_v.at[k]], add=True)

    plsc.subcore_barrier()
    pltpu.sync_copy(acc.at[pl.ds(s * RPT, RPT)],
                    deg_hbm.at[c, pl.ds(s * RPT, RPT)])


IB = 16
NCHT = EP // C
NCH0 = 80
NCH1 = 80


@functools.partial(
    pl.kernel,
    out_type=jax.ShapeDtypeStruct((NC, NP, D), jnp.float32),
    mesh=_mesh,
    scratch_types=[
        pltpu.VMEM((IB, C), jnp.int32),
        pltpu.VMEM((IB, C), jnp.int32),
        pltpu.VMEM((C, D), jnp.float32),
        pltpu.VMEM((C, D), jnp.float32),
        pltpu.VMEM_SHARED((NP, D), jnp.float32),
        pltpu.SemaphoreType.DMA,
        pltpu.SemaphoreType.DMA,
    ],
)
def _spmm_sc(src_hbm, gidx_hbm, sidx_hbm, out_hbm, gi_v, si_v, buf0, buf1,
             acc, sem0, sem1):
    c = lax.axis_index("c")
    s = lax.axis_index("s")

    _zero_fill(buf0, C, D)
    for t in range(RPT // C):
        pltpu.sync_copy(buf0, acc.at[pl.ds(s * RPT + t * C, C)])
    plsc.subcore_barrier()

    base = jnp.where(c == 0, s * NCH0, NS * NCH0 + s * NCH1)

    def run(nblocks):
        @pl.loop(0, nblocks)
        def _(b):
            start = base + b * IB
            pltpu.sync_copy(gidx_hbm.at[pl.ds(start, IB)], gi_v)
            pltpu.sync_copy(sidx_hbm.at[pl.ds(start, IB)], si_v)
            pltpu.async_copy(src_hbm.at[gi_v.at[0]], buf0, sem0)
            pltpu.async_copy(src_hbm.at[gi_v.at[1]], buf1, sem1)

            @pl.loop(0, IB, step=2)
            def _(k):
                pltpu.make_async_copy(src_hbm.at[gi_v.at[k]], buf0, sem0).wait()
                pltpu.sync_copy(buf0, acc.at[si_v.at[k]], add=True)

                @pl.when(k + 2 < IB)
                def _():
                    pltpu.async_copy(src_hbm.at[gi_v.at[k + 2]], buf0, sem0)

                pltpu.make_async_copy(src_hbm.at[gi_v.at[k + 1]], buf1,
                                      sem1).wait()
                pltpu.sync_copy(buf1, acc.at[si_v.at[k + 1]], add=True)

                @pl.when(k + 3 < IB)
                def _():
                    pltpu.async_copy(src_hbm.at[gi_v.at[k + 3]], buf1, sem1)

    @pl.when(c == 0)
    def _():
        run(NCH0 // IB)

    @pl.when(c != 0)
    def _():
        run(NCH1 // IB)

    plsc.subcore_barrier()
    pltpu.sync_copy(acc.at[pl.ds(s * RPT, RPT)],
                    out_hbm.at[c, pl.ds(s * RPT, RPT)])


_BLK = 1280
_GRID = NP // _BLK


def _prep_body(dop, dip, x, do16, di16, s1, s2):
    dgo = dop[0] + dop[1]
    dgi = dip[0] + dip[1]
    do = jnp.where(dgo > 0, lax.rsqrt(dgo), 0.0)
    di = jnp.where(dgi > 0, lax.rsqrt(dgi), 0.0)
    do16[...] = do[:, :16]
    di16[...] = di[:, :16]
    s1[...] = x[...] * di[:, :1]
    s2[...] = x[...] * do[:, :1]


def _tc_prep(dop, dip, xp):
    f32 = jnp.float32
    return pl.pallas_call(
        _prep_body,
        grid=(_GRID,),
        in_specs=[
            pl.BlockSpec((NC, _BLK, D), lambda i: (0, i, 0)),
            pl.BlockSpec((NC, _BLK, D), lambda i: (0, i, 0)),
            pl.BlockSpec((_BLK, D), lambda i: (i, 0)),
        ],
        out_specs=[
            pl.BlockSpec((_BLK, 16), lambda i: (i, 0)),
            pl.BlockSpec((_BLK, 16), lambda i: (i, 0)),
            pl.BlockSpec((_BLK, D), lambda i: (i, 0)),
            pl.BlockSpec((_BLK, D), lambda i: (i, 0)),
        ],
        out_shape=[
            jax.ShapeDtypeStruct((NP, 16), f32),
            jax.ShapeDtypeStruct((NP, 16), f32),
            jax.ShapeDtypeStruct((NP, D), f32),
            jax.ShapeDtypeStruct((NP, D), f32),
        ],
    )(dop, dip, xp)


def _mid_body(up, a16, b16, y, nxt):
    u = up[0] + up[1]
    yv = u * a16[:, :1]
    y[...] = yv
    nxt[...] = yv * b16[:, :1]


def _tc_mid(up, a16, b16):
    f32 = jnp.float32
    return pl.pallas_call(
        _mid_body,
        grid=(_GRID,),
        in_specs=[
            pl.BlockSpec((NC, _BLK, D), lambda i: (0, i, 0)),
            pl.BlockSpec((_BLK, 16), lambda i: (i, 0)),
            pl.BlockSpec((_BLK, 16), lambda i: (i, 0)),
        ],
        out_specs=[
            pl.BlockSpec((_BLK, D), lambda i: (i, 0)),
            pl.BlockSpec((_BLK, D), lambda i: (i, 0)),
        ],
        out_shape=[
            jax.ShapeDtypeStruct((NP, D), f32),
            jax.ShapeDtypeStruct((NP, D), f32),
        ],
    )(up, a16, b16)


def _final_body(x, y1, y2, v1p, v2p, di16, W_sd, b_sd, W_ds, b_ds,
                W_sd0, b_sd0, W_ds0, b_ds0, out):
    div = di16[:, :1]
    yt1 = (v1p[0] + v1p[1]) * div
    yt2 = (v2p[0] + v2p[1]) * div
    G0 = 0.5 * (W_sd0[...] + W_ds0[...])
    G1 = 0.5 * W_sd[0]
    H1 = 0.5 * W_ds[0]
    G2 = 0.25 * W_sd[1]
    H2 = 0.25 * W_ds[1]
    bt = (0.5 * (b_sd[0:1] + b_ds[0:1] + b_sd0[...] + b_ds0[...])
          + 0.25 * (b_sd[1:2] + b_ds[1:2]))
    f32 = jnp.float32
    acc = jnp.dot(x[...], G0, preferred_element_type=f32)
    acc += jnp.dot(y1[...], G1, preferred_element_type=f32)
    acc += jnp.dot(yt1, H1, preferred_element_type=f32)
    acc += jnp.dot(y2[...], G2, preferred_element_type=f32)
    acc += jnp.dot(yt2, H2, preferred_element_type=f32)
    out[...] = acc + bt


def _tc_final(xp, y1, y2, v1p, v2p, di16, W_sd, b_sd, W_ds, b_ds,
              W_sd0, b_sd0, W_ds0, b_ds0):
    wspec = pl.BlockSpec((2, D, D), lambda i: (0, 0, 0))
    bspec = pl.BlockSpec((2, D), lambda i: (0, 0))
    w0spec = pl.BlockSpec((D, D), lambda i: (0, 0))
    b0spec = pl.BlockSpec((1, D), lambda i: (0, 0))
    return pl.pallas_call(
        _final_body,
        grid=(_GRID,),
        in_specs=[
            pl.BlockSpec((_BLK, D), lambda i: (i, 0)),
            pl.BlockSpec((_BLK, D), lambda i: (i, 0)),
            pl.BlockSpec((_BLK, D), lambda i: (i, 0)),
            pl.BlockSpec((NC, _BLK, D), lambda i: (0, i, 0)),
            pl.BlockSpec((NC, _BLK, D), lambda i: (0, i, 0)),
            pl.BlockSpec((_BLK, 16), lambda i: (i, 0)),
            wspec, bspec, wspec, bspec,
            w0spec, b0spec, w0spec, b0spec,
        ],
        out_specs=pl.BlockSpec((_BLK, D), lambda i: (i, 0)),
        out_shape=jax.ShapeDtypeStruct((NP, D), jnp.float32),
    )(xp, y1, y2, v1p, v2p, di16, W_sd, b_sd, W_ds, b_ds,
      W_sd0, b_sd0, W_ds0, b_ds0)


def kernel(x, edge_index, W_sd, b_sd, W_ds, b_ds, W_sd0, b_sd0, W_ds0, b_ds0):
    row = edge_index[0].astype(jnp.int32)
    col = edge_index[1].astype(jnp.int32)
    pad = jnp.full((EP - E,), N, jnp.int32)
    rowf = jnp.concatenate([row, pad]).reshape(NCHT, C)
    colf = jnp.concatenate([col, pad]).reshape(NCHT, C)
    rowp = rowf.reshape(NW, NCHUNK, C)
    colp = colf.reshape(NW, NCHUNK, C)
    xp = jnp.pad(x, ((0, NP - N), (0, 0)))

    dop = _deg_sc(rowp)
    dip = _deg_sc(colp)
    do16, di16, s1, s2 = _tc_prep(dop, dip, xp)

    u1p = _spmm_sc(s1, colf, rowf)
    v1p = _spmm_sc(s2, rowf, colf)
    y1, s3 = _tc_mid(u1p, do16, di16)
    u2p = _spmm_sc(s3, colf, rowf)
    y2, s4 = _tc_mid(u2p, do16, do16)
    v2p = _spmm_sc(s4, rowf, colf)

    total = _tc_final(xp, y1, y2, v1p, v2p, di16, W_sd, b_sd, W_ds, b_ds,
                      W_sd0, b_sd0.reshape(1, D), W_ds0, b_ds0.reshape(1, D))
    return total[:N]


# speedup vs baseline: 1.2429x; 1.2429x over previous
import functools

import jax
import jax.numpy as jnp
from jax import lax
from jax.experimental import pallas as pl
from jax.experimental.pallas import tpu as pltpu
from jax.experimental.pallas import tpu_sc as plsc

N = 10000
D = 128
E = 320000

NC = 2
NS = 16
NW = NC * NS
C = 128
NCHUNK = 80
EPT = NCHUNK * C
EP = NW * EPT
NP = 10240
RPT = NP // NS

_mesh = plsc.VectorSubcoreMesh(core_axis_name="c", subcore_axis_name="s")


def _zero_fill(buf, rows, width):
    z = jnp.zeros((16,), jnp.float32)

    @pl.loop(0, rows)
    def _(i):
        for j in range(width // 16):
            buf[i, pl.ds(j * 16, 16)] = z


def _fill_ones(buf, rows, width):
    o = jnp.ones((16,), jnp.float32)

    @pl.loop(0, rows)
    def _(i):
        for j in range(width // 16):
            buf[i, pl.ds(j * 16, 16)] = o


@functools.partial(
    pl.kernel,
    out_type=jax.ShapeDtypeStruct((NC, NP, D), jnp.float32),
    mesh=_mesh,
    scratch_types=[
        pltpu.VMEM((NCHUNK, C), jnp.int32),
        pltpu.VMEM((C, D), jnp.float32),
        pltpu.VMEM_SHARED((NP, D), jnp.float32),
    ],
)
def _deg_sc(idx_hbm, deg_hbm, ix_v, ones_v, acc):
    c = lax.axis_index("c")
    s = lax.axis_index("s")
    wid = c * NS + s

    pltpu.sync_copy(idx_hbm.at[wid], ix_v)

    _zero_fill(ones_v, C, D)
    for t in range(RPT // C):
        pltpu.sync_copy(ones_v, acc.at[pl.ds(s * RPT + t * C, C)])
    _fill_ones(ones_v, C, D)
    plsc.subcore_barrier()

    @pl.loop(0, NCHUNK)
    def _(k):
        pltpu.sync_copy(ones_v, acc.at[ix_v.at[k]], add=True)

    plsc.subcore_barrier()
    pltpu.sync_copy(acc.at[pl.ds(s * RPT, RPT)],
                    deg_hbm.at[c, pl.ds(s * RPT, RPT)])


IB = 16
NBLK = NCHUNK // IB


@functools.partial(
    pl.kernel,
    out_type=jax.ShapeDtypeStruct((NC, NP, D), jnp.float32),
    mesh=_mesh,
    scratch_types=[
        pltpu.VMEM((IB, C), jnp.int32),
        pltpu.VMEM((IB, C), jnp.int32),
        pltpu.VMEM((C, D), jnp.float32),
        pltpu.VMEM((C, D), jnp.float32),
        pltpu.VMEM_SHARED((NP, D), jnp.float32),
        pltpu.SemaphoreType.DMA,
        pltpu.SemaphoreType.DMA,
        pltpu.SemaphoreType.DMA,
        pltpu.SemaphoreType.DMA,
    ],
)
def _spmm_sc(src_hbm, gidx_hbm, sidx_hbm, out_hbm, gi_v, si_v, buf0, buf1,
             acc, semg0, semg1, sems0, sems1):
    c = lax.axis_index("c")
    s = lax.axis_index("s")
    wid = c * NS + s

    _zero_fill(buf0, C, D)
    for t in range(RPT // C):
        pltpu.sync_copy(buf0, acc.at[pl.ds(s * RPT + t * C, C)])
    plsc.subcore_barrier()

    @pl.loop(0, NBLK)
    def _(b):
        pltpu.sync_copy(gidx_hbm.at[wid, pl.ds(b * IB, IB)], gi_v)
        pltpu.sync_copy(sidx_hbm.at[wid, pl.ds(b * IB, IB)], si_v)
        pltpu.async_copy(src_hbm.at[gi_v.at[0]], buf0, semg0)
        pltpu.async_copy(src_hbm.at[gi_v.at[1]], buf1, semg1)

        @pl.loop(0, IB, step=2)
        def _(k):
            pltpu.make_async_copy(src_hbm.at[gi_v.at[k]], buf0, semg0).wait()
            pltpu.async_copy(buf0, acc.at[si_v.at[k]], sems0, add=True)

            pltpu.make_async_copy(src_hbm.at[gi_v.at[k + 1]], buf1,
                                  semg1).wait()
            pltpu.async_copy(buf1, acc.at[si_v.at[k + 1]], sems1, add=True)

            pltpu.make_async_copy(buf0, acc.at[si_v.at[k]], sems0).wait()

            @pl.when(k + 2 < IB)
            def _():
                pltpu.async_copy(src_hbm.at[gi_v.at[k + 2]], buf0, semg0)

            pltpu.make_async_copy(buf1, acc.at[si_v.at[k + 1]], sems1).wait()

            @pl.when(k + 3 < IB)
            def _():
                pltpu.async_copy(src_hbm.at[gi_v.at[k + 3]], buf1, semg1)

    plsc.subcore_barrier()
    pltpu.sync_copy(acc.at[pl.ds(s * RPT, RPT)],
                    out_hbm.at[c, pl.ds(s * RPT, RPT)])


_BLK = 1280
_GRID = NP // _BLK


def _prep_body(dop, dip, x, do16, di16, s1, s2):
    dgo = dop[0] + dop[1]
    dgi = dip[0] + dip[1]
    do = jnp.where(dgo > 0, lax.rsqrt(dgo), 0.0)
    di = jnp.where(dgi > 0, lax.rsqrt(dgi), 0.0)
    do16[...] = do[:, :16]
    di16[...] = di[:, :16]
    s1[...] = x[...] * di[:, :1]
    s2[...] = x[...] * do[:, :1]


def _tc_prep(dop, dip, xp):
    f32 = jnp.float32
    return pl.pallas_call(
        _prep_body,
        grid=(_GRID,),
        in_specs=[
            pl.BlockSpec((NC, _BLK, D), lambda i: (0, i, 0)),
            pl.BlockSpec((NC, _BLK, D), lambda i: (0, i, 0)),
            pl.BlockSpec((_BLK, D), lambda i: (i, 0)),
        ],
        out_specs=[
            pl.BlockSpec((_BLK, 16), lambda i: (i, 0)),
            pl.BlockSpec((_BLK, 16), lambda i: (i, 0)),
            pl.BlockSpec((_BLK, D), lambda i: (i, 0)),
            pl.BlockSpec((_BLK, D), lambda i: (i, 0)),
        ],
        out_shape=[
            jax.ShapeDtypeStruct((NP, 16), f32),
            jax.ShapeDtypeStruct((NP, 16), f32),
            jax.ShapeDtypeStruct((NP, D), f32),
            jax.ShapeDtypeStruct((NP, D), f32),
        ],
    )(dop, dip, xp)


def _mid_body(up, a16, b16, y, nxt):
    u = up[0] + up[1]
    yv = u * a16[:, :1]
    y[...] = yv
    nxt[...] = yv * b16[:, :1]


def _tc_mid(up, a16, b16):
    f32 = jnp.float32
    return pl.pallas_call(
        _mid_body,
        grid=(_GRID,),
        in_specs=[
            pl.BlockSpec((NC, _BLK, D), lambda i: (0, i, 0)),
            pl.BlockSpec((_BLK, 16), lambda i: (i, 0)),
            pl.BlockSpec((_BLK, 16), lambda i: (i, 0)),
        ],
        out_specs=[
            pl.BlockSpec((_BLK, D), lambda i: (i, 0)),
            pl.BlockSpec((_BLK, D), lambda i: (i, 0)),
        ],
        out_shape=[
            jax.ShapeDtypeStruct((NP, D), f32),
            jax.ShapeDtypeStruct((NP, D), f32),
        ],
    )(up, a16, b16)


def _final_body(x, y1, y2, v1p, v2p, di16, W_sd, b_sd, W_ds, b_ds,
                W_sd0, b_sd0, W_ds0, b_ds0, out):
    div = di16[:, :1]
    yt1 = (v1p[0] + v1p[1]) * div
    yt2 = (v2p[0] + v2p[1]) * div
    G0 = 0.5 * (W_sd0[...] + W_ds0[...])
    G1 = 0.5 * W_sd[0]
    H1 = 0.5 * W_ds[0]
    G2 = 0.25 * W_sd[1]
    H2 = 0.25 * W_ds[1]
    bt = (0.5 * (b_sd[0:1] + b_ds[0:1] + b_sd0[...] + b_ds0[...])
          + 0.25 * (b_sd[1:2] + b_ds[1:2]))
    f32 = jnp.float32
    acc = jnp.dot(x[...], G0, preferred_element_type=f32)
    acc += jnp.dot(y1[...], G1, preferred_element_type=f32)
    acc += jnp.dot(yt1, H1, preferred_element_type=f32)
    acc += jnp.dot(y2[...], G2, preferred_element_type=f32)
    acc += jnp.dot(yt2, H2, preferred_element_type=f32)
    out[...] = acc + bt


def _tc_final(xp, y1, y2, v1p, v2p, di16, W_sd, b_sd, W_ds, b_ds,
              W_sd0, b_sd0, W_ds0, b_ds0):
    wspec = pl.BlockSpec((2, D, D), lambda i: (0, 0, 0))
    bspec = pl.BlockSpec((2, D), lambda i: (0, 0))
    w0spec = pl.BlockSpec((D, D), lambda i: (0, 0))
    b0spec = pl.BlockSpec((1, D), lambda i: (0, 0))
    return pl.pallas_call(
        _final_body,
        grid=(_GRID,),
        in_specs=[
            pl.BlockSpec((_BLK, D), lambda i: (i, 0)),
            pl.BlockSpec((_BLK, D), lambda i: (i, 0)),
            pl.BlockSpec((_BLK, D), lambda i: (i, 0)),
            pl.BlockSpec((NC, _BLK, D), lambda i: (0, i, 0)),
            pl.BlockSpec((NC, _BLK, D), lambda i: (0, i, 0)),
            pl.BlockSpec((_BLK, 16), lambda i: (i, 0)),
            wspec, bspec, wspec, bspec,
            w0spec, b0spec, w0spec, b0spec,
        ],
        out_specs=pl.BlockSpec((_BLK, D), lambda i: (i, 0)),
        out_shape=jax.ShapeDtypeStruct((NP, D), jnp.float32),
    )(xp, y1, y2, v1p, v2p, di16, W_sd, b_sd, W_ds, b_ds,
      W_sd0, b_sd0, W_ds0, b_ds0)


def kernel(x, edge_index, W_sd, b_sd, W_ds, b_ds, W_sd0, b_sd0, W_ds0, b_ds0):
    row = edge_index[0].astype(jnp.int32)
    col = edge_index[1].astype(jnp.int32)
    pad = jnp.full((EP - E,), N, jnp.int32)
    rowp = jnp.concatenate([row, pad]).reshape(NW, NCHUNK, C)
    colp = jnp.concatenate([col, pad]).reshape(NW, NCHUNK, C)
    xp = jnp.pad(x, ((0, NP - N), (0, 0)))

    dop = _deg_sc(rowp)
    dip = _deg_sc(colp)
    do16, di16, s1, s2 = _tc_prep(dop, dip, xp)

    u1p = _spmm_sc(s1, colp, rowp)
    v1p = _spmm_sc(s2, rowp, colp)
    y1, s3 = _tc_mid(u1p, do16, di16)
    u2p = _spmm_sc(s3, colp, rowp)
    y2, s4 = _tc_mid(u2p, do16, do16)
    v2p = _spmm_sc(s4, rowp, colp)

    total = _tc_final(xp, y1, y2, v1p, v2p, di16, W_sd, b_sd, W_ds, b_ds,
                      W_sd0, b_sd0.reshape(1, D), W_ds0, b_ds0.reshape(1, D))
    return total[:N]


# speedup vs baseline: 1.3696x; 1.1019x over previous
import functools

import jax
import jax.numpy as jnp
from jax import lax
from jax.experimental import pallas as pl
from jax.experimental.pallas import tpu as pltpu
from jax.experimental.pallas import tpu_sc as plsc

N = 10000
D = 128
E = 320000

NC = 2
NS = 16
NW = NC * NS
C = 128
NCHUNK = 80
EPT = NCHUNK * C
EP = NW * EPT
NP = 10240
RPT = NP // NS

_mesh = plsc.VectorSubcoreMesh(core_axis_name="c", subcore_axis_name="s")


def _zero_fill(buf, rows, width):
    z = jnp.zeros((16,), jnp.float32)

    @pl.loop(0, rows)
    def _(i):
        for j in range(width // 16):
            buf[i, pl.ds(j * 16, 16)] = z


def _fill_ones(buf, rows, width):
    o = jnp.ones((16,), jnp.float32)

    @pl.loop(0, rows)
    def _(i):
        for j in range(width // 16):
            buf[i, pl.ds(j * 16, 16)] = o


@functools.partial(
    pl.kernel,
    out_type=jax.ShapeDtypeStruct((NC, NP, D), jnp.float32),
    mesh=_mesh,
    scratch_types=[
        pltpu.VMEM((NCHUNK, C), jnp.int32),
        pltpu.VMEM((C, D), jnp.float32),
        pltpu.VMEM_SHARED((NP, D), jnp.float32),
    ],
)
def _deg_sc(idx_hbm, deg_hbm, ix_v, ones_v, acc):
    c = lax.axis_index("c")
    s = lax.axis_index("s")
    wid = c * NS + s

    pltpu.sync_copy(idx_hbm.at[wid], ix_v)

    _zero_fill(ones_v, C, D)
    for t in range(RPT // C):
        pltpu.sync_copy(ones_v, acc.at[pl.ds(s * RPT + t * C, C)])
    _fill_ones(ones_v, C, D)
    plsc.subcore_barrier()

    @pl.loop(0, NCHUNK)
    def _(k):
        pltpu.sync_copy(ones_v, acc.at[ix_v.at[k]], add=True)

    plsc.subcore_barrier()
    pltpu.sync_copy(acc.at[pl.ds(s * RPT, RPT)],
                    deg_hbm.at[c, pl.ds(s * RPT, RPT)])


IB = 40
NCHMAX = 120
NCHMIN = 40


@functools.partial(
    pl.kernel,
    out_type=jax.ShapeDtypeStruct((NC, NP, D), jnp.float32),
    mesh=_mesh,
    scratch_types=[
        pltpu.VMEM((IB, C), jnp.int32),
        pltpu.VMEM((IB, C), jnp.int32),
        pltpu.VMEM((C, D), jnp.float32),
        pltpu.VMEM((C, D), jnp.float32),
        pltpu.VMEM_SHARED((NP, D), jnp.float32),
        pltpu.SemaphoreType.DMA,
        pltpu.SemaphoreType.DMA,
    ],
)
def _spmm_sc(src_hbm, gidx_hbm, sidx_hbm, out_hbm, gi_v, si_v, buf0, buf1,
             acc, sem0, sem1):
    c = lax.axis_index("c")
    s = lax.axis_index("s")
    wid = c * NS + s

    _zero_fill(buf0, C, D)
    for t in range(RPT // C):
        pltpu.sync_copy(buf0, acc.at[pl.ds(s * RPT + t * C, C)])
    plsc.subcore_barrier()

    nblk = jnp.where(c == 0, NCHMAX // IB, NCHMIN // IB)

    @pl.loop(0, nblk)
    def _(b):
        pltpu.sync_copy(gidx_hbm.at[wid, pl.ds(b * IB, IB)], gi_v)
        pltpu.sync_copy(sidx_hbm.at[wid, pl.ds(b * IB, IB)], si_v)
        pltpu.async_copy(src_hbm.at[gi_v.at[0]], buf0, sem0)
        pltpu.async_copy(src_hbm.at[gi_v.at[1]], buf1, sem1)

        @pl.loop(0, IB, step=2)
        def _(k):
            pltpu.make_async_copy(src_hbm.at[gi_v.at[k]], buf0, sem0).wait()
            pltpu.sync_copy(buf0, acc.at[si_v.at[k]], add=True)

            @pl.when(k + 2 < IB)
            def _():
                pltpu.async_copy(src_hbm.at[gi_v.at[k + 2]], buf0, sem0)

            pltpu.make_async_copy(src_hbm.at[gi_v.at[k + 1]], buf1,
                                  sem1).wait()
            pltpu.sync_copy(buf1, acc.at[si_v.at[k + 1]], add=True)

            @pl.when(k + 3 < IB)
            def _():
                pltpu.async_copy(src_hbm.at[gi_v.at[k + 3]], buf1, sem1)

    plsc.subcore_barrier()
    pltpu.sync_copy(acc.at[pl.ds(s * RPT, RPT)],
                    out_hbm.at[c, pl.ds(s * RPT, RPT)])


_BLK = 1280
_GRID = NP // _BLK


def _prep_body(dop, dip, x, do16, di16, s1, s2):
    dgo = dop[0] + dop[1]
    dgi = dip[0] + dip[1]
    do = jnp.where(dgo > 0, lax.rsqrt(dgo), 0.0)
    di = jnp.where(dgi > 0, lax.rsqrt(dgi), 0.0)
    do16[...] = do[:, :16]
    di16[...] = di[:, :16]
    s1[...] = x[...] * di[:, :1]
    s2[...] = x[...] * do[:, :1]


def _tc_prep(dop, dip, xp):
    f32 = jnp.float32
    return pl.pallas_call(
        _prep_body,
        grid=(_GRID,),
        in_specs=[
            pl.BlockSpec((NC, _BLK, D), lambda i: (0, i, 0)),
            pl.BlockSpec((NC, _BLK, D), lambda i: (0, i, 0)),
            pl.BlockSpec((_BLK, D), lambda i: (i, 0)),
        ],
        out_specs=[
            pl.BlockSpec((_BLK, 16), lambda i: (i, 0)),
            pl.BlockSpec((_BLK, 16), lambda i: (i, 0)),
            pl.BlockSpec((_BLK, D), lambda i: (i, 0)),
            pl.BlockSpec((_BLK, D), lambda i: (i, 0)),
        ],
        out_shape=[
            jax.ShapeDtypeStruct((NP, 16), f32),
            jax.ShapeDtypeStruct((NP, 16), f32),
            jax.ShapeDtypeStruct((NP, D), f32),
            jax.ShapeDtypeStruct((NP, D), f32),
        ],
    )(dop, dip, xp)


def _mid_body(up, a16, b16, y, nxt):
    u = up[0] + up[1]
    yv = u * a16[:, :1]
    y[...] = yv
    nxt[...] = yv * b16[:, :1]


def _tc_mid(up, a16, b16):
    f32 = jnp.float32
    return pl.pallas_call(
        _mid_body,
        grid=(_GRID,),
        in_specs=[
            pl.BlockSpec((NC, _BLK, D), lambda i: (0, i, 0)),
            pl.BlockSpec((_BLK, 16), lambda i: (i, 0)),
            pl.BlockSpec((_BLK, 16), lambda i: (i, 0)),
        ],
        out_specs=[
            pl.BlockSpec((_BLK, D), lambda i: (i, 0)),
            pl.BlockSpec((_BLK, D), lambda i: (i, 0)),
        ],
        out_shape=[
            jax.ShapeDtypeStruct((NP, D), f32),
            jax.ShapeDtypeStruct((NP, D), f32),
        ],
    )(up, a16, b16)


def _final_body(x, y1, y2, v1p, v2p, di16, W_sd, b_sd, W_ds, b_ds,
                W_sd0, b_sd0, W_ds0, b_ds0, out):
    div = di16[:, :1]
    yt1 = (v1p[0] + v1p[1]) * div
    yt2 = (v2p[0] + v2p[1]) * div
    G0 = 0.5 * (W_sd0[...] + W_ds0[...])
    G1 = 0.5 * W_sd[0]
    H1 = 0.5 * W_ds[0]
    G2 = 0.25 * W_sd[1]
    H2 = 0.25 * W_ds[1]
    bt = (0.5 * (b_sd[0:1] + b_ds[0:1] + b_sd0[...] + b_ds0[...])
          + 0.25 * (b_sd[1:2] + b_ds[1:2]))
    f32 = jnp.float32
    acc = jnp.dot(x[...], G0, preferred_element_type=f32)
    acc += jnp.dot(y1[...], G1, preferred_element_type=f32)
    acc += jnp.dot(yt1, H1, preferred_element_type=f32)
    acc += jnp.dot(y2[...], G2, preferred_element_type=f32)
    acc += jnp.dot(yt2, H2, preferred_element_type=f32)
    out[...] = acc + bt


def _tc_final(xp, y1, y2, v1p, v2p, di16, W_sd, b_sd, W_ds, b_ds,
              W_sd0, b_sd0, W_ds0, b_ds0):
    wspec = pl.BlockSpec((2, D, D), lambda i: (0, 0, 0))
    bspec = pl.BlockSpec((2, D), lambda i: (0, 0))
    w0spec = pl.BlockSpec((D, D), lambda i: (0, 0))
    b0spec = pl.BlockSpec((1, D), lambda i: (0, 0))
    return pl.pallas_call(
        _final_body,
        grid=(_GRID,),
        in_specs=[
            pl.BlockSpec((_BLK, D), lambda i: (i, 0)),
            pl.BlockSpec((_BLK, D), lambda i: (i, 0)),
            pl.BlockSpec((_BLK, D), lambda i: (i, 0)),
            pl.BlockSpec((NC, _BLK, D), lambda i: (0, i, 0)),
            pl.BlockSpec((NC, _BLK, D), lambda i: (0, i, 0)),
            pl.BlockSpec((_BLK, 16), lambda i: (i, 0)),
            wspec, bspec, wspec, bspec,
            w0spec, b0spec, w0spec, b0spec,
        ],
        out_specs=pl.BlockSpec((_BLK, D), lambda i: (i, 0)),
        out_shape=jax.ShapeDtypeStruct((NP, D), jnp.float32),
    )(xp, y1, y2, v1p, v2p, di16, W_sd, b_sd, W_ds, b_ds,
      W_sd0, b_sd0, W_ds0, b_ds0)


def kernel(x, edge_index, W_sd, b_sd, W_ds, b_ds, W_sd0, b_sd0, W_ds0, b_ds0):
    row = edge_index[0].astype(jnp.int32)
    col = edge_index[1].astype(jnp.int32)
    pad = jnp.full((EP - E,), N, jnp.int32)
    rowflat = jnp.concatenate([row, pad]).reshape(EP // C, C)
    colflat = jnp.concatenate([col, pad]).reshape(EP // C, C)
    rowp = rowflat.reshape(NW, NCHUNK, C)
    colp = colflat.reshape(NW, NCHUNK, C)

    def _uneven(flat):
        n0 = NS * NCHMAX
        c0 = flat[:n0].reshape(NS, NCHMAX, C)
        c1 = flat[n0:].reshape(NS, NCHMIN, C)
        c1 = jnp.concatenate(
            [c1, jnp.full((NS, NCHMAX - NCHMIN, C), N, jnp.int32)], axis=1)
        return jnp.concatenate([c0, c1], axis=0)

    rowu = _uneven(rowflat)
    colu = _uneven(colflat)
    xp = jnp.pad(x, ((0, NP - N), (0, 0)))

    dop = _deg_sc(rowp)
    dip = _deg_sc(colp)
    do16, di16, s1, s2 = _tc_prep(dop, dip, xp)

    u1p = _spmm_sc(s1, colu, rowu)
    v1p = _spmm_sc(s2, rowu, colu)
    y1, s3 = _tc_mid(u1p, do16, di16)
    u2p = _spmm_sc(s3, colu, rowu)
    y2, s4 = _tc_mid(u2p, do16, do16)
    v2p = _spmm_sc(s4, rowu, colu)

    total = _tc_final(xp, y1, y2, v1p, v2p, di16, W_sd, b_sd, W_ds, b_ds,
                      W_sd0, b_sd0.reshape(1, D), W_ds0, b_ds0.reshape(1, D))
    return total[:N]
